# Initial kernel scaffold; baseline (speedup 1.0000x reference)
import jax
import jax.numpy as jnp
from jax.experimental import pallas as pl


def kernel(x, sl, table, W, b):
    raise NotImplementedError("write your pallas kernel here")


# speedup vs baseline: 11.1199x; 11.1199x over previous
import functools

import jax
import jax.numpy as jnp
from jax import lax
from jax.experimental import pallas as pl
from jax.experimental.pallas import tpu as pltpu
from jax.experimental.pallas import tpu_sc as plsc


DP = 16


def _project_table(table, r_mat):
    n, d = table.shape
    m = n * d // 128
    xr = table.reshape(m, 128)
    bm = 2000
    assert m % bm == 0

    def mm(xb, rb, ob):
        ob[...] = jnp.dot(xb[...], rb[...],
                          preferred_element_type=jnp.float32,
                          precision=jax.lax.Precision.HIGHEST)

    out = pl.pallas_call(
        mm,
        grid=(m // bm,),
        in_specs=[
            pl.BlockSpec((bm, 128), lambda i: (i, 0)),
            pl.BlockSpec((128, 64), lambda i: (0, 0)),
        ],
        out_specs=pl.BlockSpec((bm, 64), lambda i: (i, 0)),
        out_shape=jax.ShapeDtypeStruct((m, 64), jnp.float32),
    )(xr, r_mat)
    return out.reshape(n, DP)


def _sc_pool(xf, slf, p_tab, bp, b, l):
    info = plsc.get_sparse_core_info()
    nc, ns = info.num_cores, info.num_subcores
    nw = nc * ns
    rows_per_w = b // nw
    gb = 16
    chunk = gb * l
    nch = rows_per_w // gb
    ng = chunk // 128

    mesh = plsc.VectorSubcoreMesh(core_axis_name="c", subcore_axis_name="s")

    @functools.partial(
        pl.kernel,
        mesh=mesh,
        out_type=jax.ShapeDtypeStruct((b, DP), jnp.float32),
        compiler_params=pltpu.CompilerParams(use_tc_tiling_on_sc=False),
        scratch_types=[
            pltpu.VMEM((ng, 128), jnp.int32),
            pltpu.VMEM((chunk, DP), jnp.float32),
            pltpu.VMEM((rows_per_w,), jnp.float32),
            pltpu.VMEM((rows_per_w, DP), jnp.float32),
            pltpu.VMEM((DP,), jnp.float32),
            pltpu.SemaphoreType.DMA,
        ],
    )
    def body(xf_hbm, sl_hbm, p_hbm, bp_hbm, out_hbm,
             idx_v, rows_v, sl_v, out_v, bp_v, sem):
        wid = lax.axis_index("s") * nc + lax.axis_index("c")
        b0 = wid * rows_per_w
        pltpu.sync_copy(sl_hbm.at[pl.ds(b0, rows_per_w)], sl_v)
        pltpu.sync_copy(bp_hbm, bp_v)
        bias = bp_v[...]

        def chunk_body(g, carry):
            row0 = (b0 * l) // 128 + g * ng
            pltpu.sync_copy(xf_hbm.at[pl.ds(row0, ng)], idx_v)
            copies = [
                pltpu.async_copy(p_hbm.at[idx_v.at[j]],
                                 rows_v.at[pl.ds(j * 128, 128)], sem)
                for j in range(ng)
            ]
            for c in copies:
                c.wait()
            sv = sl_v[pl.ds(g * gb, 16)]
            for r in range(gb):
                def lbody(j, accs, r=r):
                    a0, a1, a2, a3 = accs
                    base = r * l + j * 4
                    return (a0 + rows_v[base], a1 + rows_v[base + 1],
                            a2 + rows_v[base + 2], a3 + rows_v[base + 3])
                z = jnp.zeros((DP,), jnp.float32)
                a0, a1, a2, a3 = lax.fori_loop(0, l // 4, lbody, (z, z, z, z))
                acc = (a0 + a1) + (a2 + a3)
                out_v[g * gb + r] = acc / sv[r] + bias
            return carry

        lax.fori_loop(0, nch, chunk_body, 0)
        pltpu.sync_copy(out_v, out_hbm.at[pl.ds(b0, rows_per_w)])

    return body(xf, slf, p_tab, bp)


def kernel(x, sl, table, W, b):
    bsz, l = x.shape
    n_cls = W.shape[0]
    wp = jnp.zeros((DP, table.shape[1]), jnp.float32).at[:n_cls].set(W)
    r_mat = jnp.kron(jnp.eye(4, dtype=jnp.float32), wp.T)
    p_tab = _project_table(table, r_mat)
    bp = jnp.zeros((DP,), jnp.float32).at[:n_cls].set(b)
    s = _sc_pool(x.reshape(bsz * l // 128, 128), sl.astype(jnp.float32),
                 p_tab, bp, bsz, l)
    return s[:, :n_cls]


# speedup vs baseline: 12.5289x; 1.1267x over previous
import functools

import jax
import jax.numpy as jnp
from jax import lax
from jax.experimental import pallas as pl
from jax.experimental.pallas import tpu as pltpu
from jax.experimental.pallas import tpu_sc as plsc


DP = 16


def _project_table(table, r_mat):
    n, d = table.shape
    m = n * d // 128
    xr = table.reshape(m, 128)
    bm = 2000
    assert m % bm == 0

    def mm(xb, rb, ob):
        ob[...] = jnp.dot(xb[...], rb[...],
                          preferred_element_type=jnp.float32,
                          precision=jax.lax.Precision.HIGHEST)

    out = pl.pallas_call(
        mm,
        grid=(m // bm,),
        in_specs=[
            pl.BlockSpec((bm, 128), lambda i: (i, 0)),
            pl.BlockSpec((128, 64), lambda i: (0, 0)),
        ],
        out_specs=pl.BlockSpec((bm, 64), lambda i: (i, 0)),
        out_shape=jax.ShapeDtypeStruct((m, 64), jnp.float32),
    )(xr, r_mat)
    return out.reshape(n, DP)


def _sc_pool(xf, slf, p_tab, bp, b, l):
    info = plsc.get_sparse_core_info()
    nc, ns = info.num_cores, info.num_subcores
    nw = nc * ns
    rows_per_w = b // nw
    gb = 16
    chunk = gb * l
    nch = rows_per_w // gb
    ng = chunk // 128

    mesh = plsc.VectorSubcoreMesh(core_axis_name="c", subcore_axis_name="s")

    @functools.partial(
        pl.kernel,
        mesh=mesh,
        out_type=jax.ShapeDtypeStruct((b, DP), jnp.float32),
        compiler_params=pltpu.CompilerParams(use_tc_tiling_on_sc=False),
        scratch_types=[
            pltpu.VMEM((ng, 128), jnp.int32),
            pltpu.VMEM((ng, 128), jnp.int32),
            pltpu.VMEM((chunk, DP), jnp.float32),
            pltpu.VMEM((chunk, DP), jnp.float32),
            pltpu.VMEM((rows_per_w,), jnp.float32),
            pltpu.VMEM((rows_per_w, DP), jnp.float32),
            pltpu.VMEM((DP,), jnp.float32),
            pltpu.SemaphoreType.DMA,
            pltpu.SemaphoreType.DMA,
        ],
    )
    def body(xf_hbm, sl_hbm, p_hbm, bp_hbm, out_hbm,
             idx0, idx1, rows0, rows1, sl_v, out_v, bp_v, sem0, sem1):
        wid = lax.axis_index("s") * nc + lax.axis_index("c")
        b0 = wid * rows_per_w
        xrow0 = (b0 * l) // 128
        pltpu.sync_copy(sl_hbm.at[pl.ds(b0, rows_per_w)], sl_v)
        pltpu.sync_copy(bp_hbm, bp_v)
        bias = bp_v[...]

        def fire(g, idxb, rowsb, semb):
            pltpu.sync_copy(xf_hbm.at[pl.ds(xrow0 + g * ng, ng)], idxb)
            for j in range(ng):
                pltpu.async_copy(p_hbm.at[idxb.at[j]],
                                 rowsb.at[pl.ds(j * 128, 128)], semb)

        def drain(rowsb, semb):
            pltpu.make_async_copy(p_hbm.at[pl.ds(0, chunk)], rowsb,
                                  semb).wait()

        def accum(g, rowsb):
            sv = sl_v[pl.ds(g * gb, 16)]
            for r in range(gb):
                def lbody(j, accs, r=r):
                    base = r * l + j * 8
                    return tuple(accs[k] + rowsb[base + k] for k in range(8))
                z = jnp.zeros((DP,), jnp.float32)
                accs = lax.fori_loop(0, l // 8, lbody, (z,) * 8)
                acc = (((accs[0] + accs[1]) + (accs[2] + accs[3]))
                       + ((accs[4] + accs[5]) + (accs[6] + accs[7])))
                out_v[g * gb + r] = acc / sv[r] + bias

        fire(0, idx0, rows0, sem0)

        def pair_body(i, carry):
            g0 = i * 2
            fire(g0 + 1, idx1, rows1, sem1)
            drain(rows0, sem0)
            accum(g0, rows0)

            @pl.when(g0 + 2 < nch)
            def _():
                fire(g0 + 2, idx0, rows0, sem0)

            drain(rows1, sem1)
            accum(g0 + 1, rows1)
            return carry

        lax.fori_loop(0, nch // 2, pair_body, 0)
        pltpu.sync_copy(out_v, out_hbm.at[pl.ds(b0, rows_per_w)])

    return body(xf, slf, p_tab, bp)


def kernel(x, sl, table, W, b):
    bsz, l = x.shape
    n_cls = W.shape[0]
    wp = jnp.zeros((DP, table.shape[1]), jnp.float32).at[:n_cls].set(W)
    r_mat = jnp.kron(jnp.eye(4, dtype=jnp.float32), wp.T)
    p_tab = _project_table(table, r_mat)
    bp = jnp.zeros((DP,), jnp.float32).at[:n_cls].set(b)
    s = _sc_pool(x.reshape(bsz * l // 128, 128), sl.astype(jnp.float32),
                 p_tab, bp, bsz, l)
    return s[:, :n_cls]
